# Initial kernel scaffold; baseline (speedup 1.0000x reference)
import jax
import jax.numpy as jnp
from jax.experimental import pallas as pl


def kernel(x, coord, lens, cls_token, fc_w, fc_b, ln1_g, ln1_b, wq, bq, wk, bk, wv, bv, wo, bo, rel_bias, ln2_g, ln2_b, w1, b1, w2, b2, lnf_g, lnf_b, head_w, head_b):
    raise NotImplementedError("write your pallas kernel here")


# speedup vs baseline: 36.9679x; 36.9679x over previous
import math

import jax
import jax.numpy as jnp
from jax.experimental import pallas as pl

_N = 2048
_IN_DIM = 768
_DIM = 512
_DEPTH = 2
_HEADS = 8
_KB = 10
_MLP = 2048
_NC = 2
_DH = _DIM // _HEADS

_NP1 = _N + 1
_TQ = 128
_NPAD = ((_NP1 + _TQ - 1) // _TQ) * _TQ
_NQT = _NPAD // _TQ
_NXT = _N // _TQ


def _lnorm(x, g, b, eps=1e-5):
    m = jnp.mean(x, axis=-1, keepdims=True)
    v = jnp.mean((x - m) ** 2, axis=-1, keepdims=True)
    return (x - m) / jnp.sqrt(v + eps) * g + b


def _embed_kernel(x_ref, w_ref, b_ref, o_ref):
    acc = jnp.dot(x_ref[...], w_ref[...], preferred_element_type=jnp.float32)
    o_ref[...] = jax.nn.gelu(acc + b_ref[...])


def _qkv_kernel(h_ref, g_ref, b_ref, wq_ref, bq_ref, wk_ref, bk_ref,
                wv_ref, bv_ref, q_ref, k_ref, v_ref):
    xn = _lnorm(h_ref[...], g_ref[...], b_ref[...])
    qf = jnp.dot(xn, wq_ref[...], preferred_element_type=jnp.float32) + bq_ref[...]
    kf = jnp.dot(xn, wk_ref[...], preferred_element_type=jnp.float32) + bk_ref[...]
    vf = jnp.dot(xn, wv_ref[...], preferred_element_type=jnp.float32) + bv_ref[...]
    for hh in range(_HEADS):
        sl = slice(hh * _DH, (hh + 1) * _DH)
        q_ref[hh] = qf[:, sl]
        k_ref[hh] = kf[:, sl]
        v_ref[hh] = vf[:, sl]


def _attn_kernel(q_ref, k_ref, v_ref, cq_ref, ck_ref, rel_ref, o_ref):
    qi = pl.program_id(1)
    q = q_ref[0]
    k = k_ref[0]
    s = jax.lax.dot_general(q, k, (((1,), (1,)), ((), ())),
                            preferred_element_type=jnp.float32)
    s = s * (1.0 / math.sqrt(_DH))

    qx = cq_ref[0, :, 0:1]
    qy = cq_ref[0, :, 1:2]
    kx = ck_ref[0:1, :]
    ky = ck_ref[1:2, :]
    dx = qx - kx
    dy = qy - ky
    dist = jnp.sqrt(dx * dx + dy * dy + 1e-12)
    fb = jnp.clip(jnp.floor(dist * _KB), 0.0, _KB - 1.0)
    bias = jnp.zeros_like(s)
    for t in range(_KB):
        bias = bias + jnp.where(fb == float(t), rel_ref[0, 0, t], 0.0)

    rows = qi * _TQ + jax.lax.broadcasted_iota(jnp.int32, (_TQ, 1), 0)
    cols = jax.lax.broadcasted_iota(jnp.int32, (1, _NPAD), 1)
    bias = jnp.where((rows == 0) | (cols == 0), 0.0, bias)
    s = s + bias
    s = jnp.where(cols > _N, -1e30, s)

    m = jnp.max(s, axis=-1, keepdims=True)
    e = jnp.exp(s - m)
    p = e / jnp.sum(e, axis=-1, keepdims=True)
    o_ref[0] = jnp.dot(p, v_ref[0], preferred_element_type=jnp.float32)


def _post_kernel(oh_ref, h_ref, wo_ref, bo_ref, g2_ref, b2_ref,
                 w1_ref, b1_ref, w2_ref, b2b_ref, out_ref):
    oc = jnp.concatenate([oh_ref[hh] for hh in range(_HEADS)], axis=1)
    h2 = h_ref[...] + jnp.dot(oc, wo_ref[...],
                              preferred_element_type=jnp.float32) + bo_ref[...]
    xn = _lnorm(h2, g2_ref[...], b2_ref[...])
    hh = jax.nn.gelu(jnp.dot(xn, w1_ref[...],
                             preferred_element_type=jnp.float32) + b1_ref[...])
    ff = jnp.dot(hh, w2_ref[...], preferred_element_type=jnp.float32) + b2b_ref[...]
    out_ref[...] = h2 + ff


def _final_kernel(h_ref, g_ref, b_ref, w_ref, bh_ref, o_ref):
    x0 = h_ref[0:1, :]
    xn = _lnorm(x0, g_ref[...], b_ref[...])
    o_ref[...] = jnp.dot(xn, w_ref[...], preferred_element_type=jnp.float32) + bh_ref[...]


def kernel(x, coord, lens, cls_token, fc_w, fc_b, ln1_g, ln1_b, wq, bq,
           wk, bk, wv, bv, wo, bo, rel_bias, ln2_g, ln2_b, w1, b1, w2, b2,
           lnf_g, lnf_b, head_w, head_b):
    f32 = jnp.float32
    x2 = x[0]

    emb = pl.pallas_call(
        _embed_kernel,
        grid=(_NXT,),
        in_specs=[
            pl.BlockSpec((_TQ, _IN_DIM), lambda i: (i, 0)),
            pl.BlockSpec((_IN_DIM, _DIM), lambda i: (0, 0)),
            pl.BlockSpec((1, _DIM), lambda i: (0, 0)),
        ],
        out_specs=pl.BlockSpec((_TQ, _DIM), lambda i: (i, 0)),
        out_shape=jax.ShapeDtypeStruct((_N, _DIM), f32),
    )(x2, fc_w, fc_b.reshape(1, _DIM))

    h = jnp.concatenate([
        cls_token.reshape(1, _DIM),
        emb,
        jnp.zeros((_NPAD - _NP1, _DIM), f32),
    ], axis=0)

    pc = jnp.concatenate([
        jnp.zeros((1, 2), f32),
        coord[0],
        jnp.zeros((_NPAD - _NP1, 2), f32),
    ], axis=0)
    cq = jnp.pad(pc, ((0, 0), (0, 6))).reshape(_NQT, _TQ, 8)
    ck = jnp.pad(pc.T, ((0, 6), (0, 0)))

    for l in range(_DEPTH):
        q, k, v = pl.pallas_call(
            _qkv_kernel,
            grid=(_NQT,),
            in_specs=[
                pl.BlockSpec((_TQ, _DIM), lambda i: (i, 0)),
                pl.BlockSpec((1, _DIM), lambda i: (0, 0)),
                pl.BlockSpec((1, _DIM), lambda i: (0, 0)),
                pl.BlockSpec((_DIM, _DIM), lambda i: (0, 0)),
                pl.BlockSpec((1, _DIM), lambda i: (0, 0)),
                pl.BlockSpec((_DIM, _DIM), lambda i: (0, 0)),
                pl.BlockSpec((1, _DIM), lambda i: (0, 0)),
                pl.BlockSpec((_DIM, _DIM), lambda i: (0, 0)),
                pl.BlockSpec((1, _DIM), lambda i: (0, 0)),
            ],
            out_specs=[
                pl.BlockSpec((_HEADS, _TQ, _DH), lambda i: (0, i, 0)),
                pl.BlockSpec((_HEADS, _TQ, _DH), lambda i: (0, i, 0)),
                pl.BlockSpec((_HEADS, _TQ, _DH), lambda i: (0, i, 0)),
            ],
            out_shape=[
                jax.ShapeDtypeStruct((_HEADS, _NPAD, _DH), f32),
                jax.ShapeDtypeStruct((_HEADS, _NPAD, _DH), f32),
                jax.ShapeDtypeStruct((_HEADS, _NPAD, _DH), f32),
            ],
        )(h, ln1_g[l].reshape(1, _DIM), ln1_b[l].reshape(1, _DIM),
          wq[l], bq[l].reshape(1, _DIM), wk[l], bk[l].reshape(1, _DIM),
          wv[l], bv[l].reshape(1, _DIM))

        rel3 = rel_bias[l].T.reshape(_HEADS, 1, _KB)
        oh = pl.pallas_call(
            _attn_kernel,
            grid=(_HEADS, _NQT),
            in_specs=[
                pl.BlockSpec((1, _TQ, _DH), lambda hh, i: (hh, i, 0)),
                pl.BlockSpec((1, _NPAD, _DH), lambda hh, i: (hh, 0, 0)),
                pl.BlockSpec((1, _NPAD, _DH), lambda hh, i: (hh, 0, 0)),
                pl.BlockSpec((1, _TQ, 8), lambda hh, i: (i, 0, 0)),
                pl.BlockSpec((8, _NPAD), lambda hh, i: (0, 0)),
                pl.BlockSpec((1, 1, _KB), lambda hh, i: (hh, 0, 0)),
            ],
            out_specs=pl.BlockSpec((1, _TQ, _DH), lambda hh, i: (hh, i, 0)),
            out_shape=jax.ShapeDtypeStruct((_HEADS, _NPAD, _DH), f32),
        )(q, k, v, cq, ck, rel3)

        h = pl.pallas_call(
            _post_kernel,
            grid=(_NQT,),
            in_specs=[
                pl.BlockSpec((_HEADS, _TQ, _DH), lambda i: (0, i, 0)),
                pl.BlockSpec((_TQ, _DIM), lambda i: (i, 0)),
                pl.BlockSpec((_DIM, _DIM), lambda i: (0, 0)),
                pl.BlockSpec((1, _DIM), lambda i: (0, 0)),
                pl.BlockSpec((1, _DIM), lambda i: (0, 0)),
                pl.BlockSpec((1, _DIM), lambda i: (0, 0)),
                pl.BlockSpec((_DIM, _MLP), lambda i: (0, 0)),
                pl.BlockSpec((1, _MLP), lambda i: (0, 0)),
                pl.BlockSpec((_MLP, _DIM), lambda i: (0, 0)),
                pl.BlockSpec((1, _DIM), lambda i: (0, 0)),
            ],
            out_specs=pl.BlockSpec((_TQ, _DIM), lambda i: (i, 0)),
            out_shape=jax.ShapeDtypeStruct((_NPAD, _DIM), f32),
        )(oh, h, wo[l], bo[l].reshape(1, _DIM),
          ln2_g[l].reshape(1, _DIM), ln2_b[l].reshape(1, _DIM),
          w1[l], b1[l].reshape(1, _MLP), w2[l], b2[l].reshape(1, _DIM))

    hw = jnp.pad(head_w, ((0, 0), (0, 128 - _NC)))
    hb = jnp.pad(head_b, (0, 128 - _NC)).reshape(1, 128)
    out = pl.pallas_call(
        _final_kernel,
        grid=(1,),
        in_specs=[
            pl.BlockSpec((8, _DIM), lambda i: (0, 0)),
            pl.BlockSpec((1, _DIM), lambda i: (0, 0)),
            pl.BlockSpec((1, _DIM), lambda i: (0, 0)),
            pl.BlockSpec((_DIM, 128), lambda i: (0, 0)),
            pl.BlockSpec((1, 128), lambda i: (0, 0)),
        ],
        out_specs=pl.BlockSpec((1, 128), lambda i: (0, 0)),
        out_shape=jax.ShapeDtypeStruct((1, 128), f32),
    )(h, lnf_g.reshape(1, _DIM), lnf_b.reshape(1, _DIM), hw, hb)

    return out[:, :_NC]


# speedup vs baseline: 64.2606x; 1.7383x over previous
import math

import jax
import jax.numpy as jnp
from jax.experimental import pallas as pl

_N = 2048
_IN_DIM = 768
_DIM = 512
_DEPTH = 2
_HEADS = 8
_KB = 10
_MLP = 2048
_NC = 2
_DH = _DIM // _HEADS

_NP1 = _N + 1
_TQ = 128
_NPAD = ((_NP1 + _TQ - 1) // _TQ) * _TQ
_NQT = _NPAD // _TQ
_NXT = _N // _TQ

_BF = jnp.bfloat16
_F32 = jnp.float32


def _lnorm(x, g, b, eps=1e-5):
    m = jnp.mean(x, axis=-1, keepdims=True)
    v = jnp.mean((x - m) ** 2, axis=-1, keepdims=True)
    return (x - m) / jnp.sqrt(v + eps) * g + b


def _embed_kernel(x_ref, w_ref, b_ref, o_ref):
    acc = jnp.dot(x_ref[...], w_ref[...], preferred_element_type=_F32)
    o_ref[...] = jax.nn.gelu(acc + b_ref[...])


def _qkv_kernel(h_ref, g_ref, b_ref, wq_ref, bq_ref, wk_ref, bk_ref,
                wv_ref, bv_ref, q_ref, k_ref, v_ref):
    xn = _lnorm(h_ref[...], g_ref[...], b_ref[...]).astype(_BF)
    q_ref[...] = (jnp.dot(xn, wq_ref[...], preferred_element_type=_F32)
                  + bq_ref[...]).astype(_BF)
    k_ref[...] = (jnp.dot(xn, wk_ref[...], preferred_element_type=_F32)
                  + bk_ref[...]).astype(_BF)
    v_ref[...] = (jnp.dot(xn, wv_ref[...], preferred_element_type=_F32)
                  + bv_ref[...]).astype(_BF)


def _layer_kernel(q_ref, k_ref, v_ref, h_ref, cq_ref, ck_ref, drel_ref,
                  wo_ref, bo_ref, g2_ref, b2_ref, w1_ref, b1_ref,
                  w2_ref, b2b_ref, out_ref):
    qi = pl.program_id(0)
    scale = 1.0 / math.sqrt(_DH)

    qx = cq_ref[0, :, 0:1]
    qy = cq_ref[0, :, 1:2]
    kx = ck_ref[0:1, :]
    ky = ck_ref[1:2, :]
    dx = qx - kx
    dy = qy - ky
    d2 = dx * dx + dy * dy
    steps = []
    for t in range(1, _KB):
        th = (t / _KB) ** 2 - 1e-12
        steps.append(jnp.where(d2 >= th, 1.0, 0.0))

    rows = qi * _TQ + jax.lax.broadcasted_iota(jnp.int32, (_TQ, 1), 0)
    cols = jax.lax.broadcasted_iota(jnp.int32, (1, _NPAD), 1)
    row0 = rows == 0
    col0 = cols == 0
    kmask = jnp.where(cols > _N, -1e30, 0.0)

    qf = q_ref[...]
    kf = k_ref[...]
    vf = v_ref[...]
    ohs = []
    for hh in range(_HEADS):
        sl = slice(hh * _DH, (hh + 1) * _DH)
        s = jax.lax.dot_general(qf[:, sl], kf[:, sl], (((1,), (1,)), ((), ())),
                                preferred_element_type=_F32)
        bias = jnp.full_like(s, drel_ref[hh, 0])
        for t in range(1, _KB):
            bias = bias + steps[t - 1] * drel_ref[hh, t]
        bias = jnp.where(row0 | col0, 0.0, bias)
        s = s * scale + bias + kmask
        m = jnp.max(s, axis=-1, keepdims=True)
        e = jnp.exp(s - m)
        ssum = jnp.sum(e, axis=-1, keepdims=True)
        oh = jnp.dot(e.astype(_BF), vf[:, sl], preferred_element_type=_F32)
        ohs.append(oh / ssum)
    oc = jnp.concatenate(ohs, axis=1).astype(_BF)

    h2 = h_ref[...] + jnp.dot(oc, wo_ref[...],
                              preferred_element_type=_F32) + bo_ref[...]
    xn = _lnorm(h2, g2_ref[...], b2_ref[...]).astype(_BF)
    hh1 = jax.nn.gelu(jnp.dot(xn, w1_ref[...],
                              preferred_element_type=_F32) + b1_ref[...])
    ff = jnp.dot(hh1.astype(_BF), w2_ref[...],
                 preferred_element_type=_F32) + b2b_ref[...]
    out_ref[...] = h2 + ff


def _final_kernel(h_ref, g_ref, b_ref, w_ref, bh_ref, o_ref):
    x0 = h_ref[0:1, :]
    xn = _lnorm(x0, g_ref[...], b_ref[...])
    o_ref[...] = jnp.dot(xn, w_ref[...], preferred_element_type=_F32) + bh_ref[...]


def kernel(x, coord, lens, cls_token, fc_w, fc_b, ln1_g, ln1_b, wq, bq,
           wk, bk, wv, bv, wo, bo, rel_bias, ln2_g, ln2_b, w1, b1, w2, b2,
           lnf_g, lnf_b, head_w, head_b):
    x2 = x[0]

    emb = pl.pallas_call(
        _embed_kernel,
        grid=(_NXT,),
        in_specs=[
            pl.BlockSpec((_TQ, _IN_DIM), lambda i: (i, 0)),
            pl.BlockSpec((_IN_DIM, _DIM), lambda i: (0, 0)),
            pl.BlockSpec((1, _DIM), lambda i: (0, 0)),
        ],
        out_specs=pl.BlockSpec((_TQ, _DIM), lambda i: (i, 0)),
        out_shape=jax.ShapeDtypeStruct((_N, _DIM), _F32),
    )(x2, fc_w, fc_b.reshape(1, _DIM))

    h = jnp.concatenate([
        cls_token.reshape(1, _DIM),
        emb,
        jnp.zeros((_NPAD - _NP1, _DIM), _F32),
    ], axis=0)

    pc = jnp.concatenate([
        jnp.zeros((1, 2), _F32),
        coord[0],
        jnp.zeros((_NPAD - _NP1, 2), _F32),
    ], axis=0)
    cq = jnp.pad(pc, ((0, 0), (0, 6))).reshape(_NQT, _TQ, 8)
    ck = jnp.pad(pc.T, ((0, 6), (0, 0)))

    for l in range(_DEPTH):
        q, k, v = pl.pallas_call(
            _qkv_kernel,
            grid=(_NQT,),
            in_specs=[
                pl.BlockSpec((_TQ, _DIM), lambda i: (i, 0)),
                pl.BlockSpec((1, _DIM), lambda i: (0, 0)),
                pl.BlockSpec((1, _DIM), lambda i: (0, 0)),
                pl.BlockSpec((_DIM, _DIM), lambda i: (0, 0)),
                pl.BlockSpec((1, _DIM), lambda i: (0, 0)),
                pl.BlockSpec((_DIM, _DIM), lambda i: (0, 0)),
                pl.BlockSpec((1, _DIM), lambda i: (0, 0)),
                pl.BlockSpec((_DIM, _DIM), lambda i: (0, 0)),
                pl.BlockSpec((1, _DIM), lambda i: (0, 0)),
            ],
            out_specs=[
                pl.BlockSpec((_TQ, _DIM), lambda i: (i, 0)),
                pl.BlockSpec((_TQ, _DIM), lambda i: (i, 0)),
                pl.BlockSpec((_TQ, _DIM), lambda i: (i, 0)),
            ],
            out_shape=[
                jax.ShapeDtypeStruct((_NPAD, _DIM), _BF),
                jax.ShapeDtypeStruct((_NPAD, _DIM), _BF),
                jax.ShapeDtypeStruct((_NPAD, _DIM), _BF),
            ],
        )(h, ln1_g[l].reshape(1, _DIM), ln1_b[l].reshape(1, _DIM),
          wq[l].astype(_BF), bq[l].reshape(1, _DIM),
          wk[l].astype(_BF), bk[l].reshape(1, _DIM),
          wv[l].astype(_BF), bv[l].reshape(1, _DIM))

        rel = rel_bias[l]
        drel = jnp.concatenate([rel[0:1], rel[1:] - rel[:-1]], axis=0).T

        h = pl.pallas_call(
            _layer_kernel,
            grid=(_NQT,),
            in_specs=[
                pl.BlockSpec((_TQ, _DIM), lambda i: (i, 0)),
                pl.BlockSpec((_NPAD, _DIM), lambda i: (0, 0)),
                pl.BlockSpec((_NPAD, _DIM), lambda i: (0, 0)),
                pl.BlockSpec((_TQ, _DIM), lambda i: (i, 0)),
                pl.BlockSpec((1, _TQ, 8), lambda i: (i, 0, 0)),
                pl.BlockSpec((8, _NPAD), lambda i: (0, 0)),
                pl.BlockSpec((_HEADS, _KB), lambda i: (0, 0)),
                pl.BlockSpec((_DIM, _DIM), lambda i: (0, 0)),
                pl.BlockSpec((1, _DIM), lambda i: (0, 0)),
                pl.BlockSpec((1, _DIM), lambda i: (0, 0)),
                pl.BlockSpec((1, _DIM), lambda i: (0, 0)),
                pl.BlockSpec((_DIM, _MLP), lambda i: (0, 0)),
                pl.BlockSpec((1, _MLP), lambda i: (0, 0)),
                pl.BlockSpec((_MLP, _DIM), lambda i: (0, 0)),
                pl.BlockSpec((1, _DIM), lambda i: (0, 0)),
            ],
            out_specs=pl.BlockSpec((_TQ, _DIM), lambda i: (i, 0)),
            out_shape=jax.ShapeDtypeStruct((_NPAD, _DIM), _F32),
        )(q, k, v, h, cq, ck, drel,
          wo[l].astype(_BF), bo[l].reshape(1, _DIM),
          ln2_g[l].reshape(1, _DIM), ln2_b[l].reshape(1, _DIM),
          w1[l].astype(_BF), b1[l].reshape(1, _MLP),
          w2[l].astype(_BF), b2[l].reshape(1, _DIM))

    hw = jnp.pad(head_w, ((0, 0), (0, 128 - _NC)))
    hb = jnp.pad(head_b, (0, 128 - _NC)).reshape(1, 128)
    out = pl.pallas_call(
        _final_kernel,
        grid=(1,),
        in_specs=[
            pl.BlockSpec((8, _DIM), lambda i: (0, 0)),
            pl.BlockSpec((1, _DIM), lambda i: (0, 0)),
            pl.BlockSpec((1, _DIM), lambda i: (0, 0)),
            pl.BlockSpec((_DIM, 128), lambda i: (0, 0)),
            pl.BlockSpec((1, 128), lambda i: (0, 0)),
        ],
        out_specs=pl.BlockSpec((1, 128), lambda i: (0, 0)),
        out_shape=jax.ShapeDtypeStruct((1, 128), _F32),
    )(h, lnf_g.reshape(1, _DIM), lnf_b.reshape(1, _DIM), hw, hb)

    return out[:, :_NC]


# speedup vs baseline: 75.6288x; 1.1769x over previous
import math

import jax
import jax.numpy as jnp
from jax.experimental import pallas as pl

_N = 2048
_IN_DIM = 768
_DIM = 512
_DEPTH = 2
_HEADS = 8
_KB = 10
_MLP = 2048
_NC = 2
_DH = _DIM // _HEADS

_NP1 = _N + 1
_TQ = 128
_NPAD = ((_NP1 + _TQ - 1) // _TQ) * _TQ
_NQT = _NPAD // _TQ
_NXT = _N // _TQ

_BF = jnp.bfloat16
_F32 = jnp.float32


def _lnorm(x, g, b, eps=1e-5):
    m = jnp.mean(x, axis=-1, keepdims=True)
    v = jnp.mean((x - m) ** 2, axis=-1, keepdims=True)
    return (x - m) / jnp.sqrt(v + eps) * g + b


def _embed_kernel(x_ref, w_ref, b_ref, o_ref):
    acc = jnp.dot(x_ref[...], w_ref[...], preferred_element_type=_F32)
    o_ref[...] = jax.nn.gelu(acc + b_ref[...])


def _qkv_kernel(h_ref, g_ref, b_ref, wq_ref, bq_ref, wk_ref, bk_ref,
                wv_ref, bv_ref, q_ref, k_ref, v_ref):
    scale = 1.0 / math.sqrt(_DH)
    xn = _lnorm(h_ref[...], g_ref[...], b_ref[...]).astype(_BF)
    q_ref[...] = ((jnp.dot(xn, wq_ref[...], preferred_element_type=_F32)
                   + bq_ref[...]) * scale).astype(_BF)
    k_ref[...] = (jnp.dot(xn, wk_ref[...], preferred_element_type=_F32)
                  + bk_ref[...]).astype(_BF)
    v_ref[...] = (jnp.dot(xn, wv_ref[...], preferred_element_type=_F32)
                  + bv_ref[...]).astype(_BF)


def _layer_kernel(q_ref, k_ref, v_ref, h_ref, cq_ref, ck_ref, drel_ref,
                  wo_ref, bo_ref, g2_ref, b2_ref, w1_ref, b1_ref,
                  w2_ref, b2b_ref, out_ref):
    qi = pl.program_id(0)

    qx = cq_ref[0, :, 0:1]
    qy = cq_ref[0, :, 1:2]
    kx = ck_ref[0:1, :]
    ky = ck_ref[1:2, :]
    dx = qx - kx
    dy = qy - ky
    d2 = dx * dx + dy * dy
    masks = [d2 >= ((t / _KB) ** 2 - 1e-12) for t in range(1, _KB)]

    rows = qi * _TQ + jax.lax.broadcasted_iota(jnp.int32, (_TQ, 1), 0)
    cols = jax.lax.broadcasted_iota(jnp.int32, (1, _NPAD), 1)
    mask0 = (rows == 0) | (cols == 0)
    kmask = jnp.where(cols > _N, -1e30, 0.0)

    qf = q_ref[...]
    kf = k_ref[...]
    vf = v_ref[...]
    ohs = []
    for hh in range(_HEADS):
        sl = slice(hh * _DH, (hh + 1) * _DH)
        s = jax.lax.dot_general(qf[:, sl], kf[:, sl], (((1,), (1,)), ((), ())),
                                preferred_element_type=_F32)
        bias = jnp.full_like(s, drel_ref[hh, 0])
        for t in range(1, _KB):
            bias = jnp.where(masks[t - 1], drel_ref[hh, t], bias)
        bias = jnp.where(mask0, 0.0, bias)
        s = s + bias + kmask
        e = jnp.exp(s)
        ssum = jnp.sum(e, axis=-1, keepdims=True)
        oh = jnp.dot(e.astype(_BF), vf[:, sl], preferred_element_type=_F32)
        ohs.append(oh * (1.0 / ssum))
    oc = jnp.concatenate(ohs, axis=1).astype(_BF)

    h2 = h_ref[...] + jnp.dot(oc, wo_ref[...],
                              preferred_element_type=_F32) + bo_ref[...]
    xn = _lnorm(h2, g2_ref[...], b2_ref[...]).astype(_BF)
    hh1 = jax.nn.gelu(jnp.dot(xn, w1_ref[...],
                              preferred_element_type=_F32) + b1_ref[...])
    ff = jnp.dot(hh1.astype(_BF), w2_ref[...],
                 preferred_element_type=_F32) + b2b_ref[...]
    out_ref[...] = h2 + ff


def _final_kernel(h_ref, g_ref, b_ref, w_ref, bh_ref, o_ref):
    x0 = h_ref[0:1, :]
    xn = _lnorm(x0, g_ref[...], b_ref[...])
    o_ref[...] = jnp.dot(xn, w_ref[...], preferred_element_type=_F32) + bh_ref[...]


def kernel(x, coord, lens, cls_token, fc_w, fc_b, ln1_g, ln1_b, wq, bq,
           wk, bk, wv, bv, wo, bo, rel_bias, ln2_g, ln2_b, w1, b1, w2, b2,
           lnf_g, lnf_b, head_w, head_b):
    x2 = x[0]

    emb = pl.pallas_call(
        _embed_kernel,
        grid=(_NXT,),
        in_specs=[
            pl.BlockSpec((_TQ, _IN_DIM), lambda i: (i, 0)),
            pl.BlockSpec((_IN_DIM, _DIM), lambda i: (0, 0)),
            pl.BlockSpec((1, _DIM), lambda i: (0, 0)),
        ],
        out_specs=pl.BlockSpec((_TQ, _DIM), lambda i: (i, 0)),
        out_shape=jax.ShapeDtypeStruct((_N, _DIM), _F32),
    )(x2, fc_w, fc_b.reshape(1, _DIM))

    h = jnp.concatenate([
        cls_token.reshape(1, _DIM),
        emb,
        jnp.zeros((_NPAD - _NP1, _DIM), _F32),
    ], axis=0)

    pc = jnp.concatenate([
        jnp.zeros((1, 2), _F32),
        coord[0],
        jnp.zeros((_NPAD - _NP1, 2), _F32),
    ], axis=0)
    cq = jnp.pad(pc, ((0, 0), (0, 6))).reshape(_NQT, _TQ, 8)
    ck = jnp.pad(pc.T, ((0, 6), (0, 0)))

    for l in range(_DEPTH):
        q, k, v = pl.pallas_call(
            _qkv_kernel,
            grid=(_NQT,),
            in_specs=[
                pl.BlockSpec((_TQ, _DIM), lambda i: (i, 0)),
                pl.BlockSpec((1, _DIM), lambda i: (0, 0)),
                pl.BlockSpec((1, _DIM), lambda i: (0, 0)),
                pl.BlockSpec((_DIM, _DIM), lambda i: (0, 0)),
                pl.BlockSpec((1, _DIM), lambda i: (0, 0)),
                pl.BlockSpec((_DIM, _DIM), lambda i: (0, 0)),
                pl.BlockSpec((1, _DIM), lambda i: (0, 0)),
                pl.BlockSpec((_DIM, _DIM), lambda i: (0, 0)),
                pl.BlockSpec((1, _DIM), lambda i: (0, 0)),
            ],
            out_specs=[
                pl.BlockSpec((_TQ, _DIM), lambda i: (i, 0)),
                pl.BlockSpec((_TQ, _DIM), lambda i: (i, 0)),
                pl.BlockSpec((_TQ, _DIM), lambda i: (i, 0)),
            ],
            out_shape=[
                jax.ShapeDtypeStruct((_NPAD, _DIM), _BF),
                jax.ShapeDtypeStruct((_NPAD, _DIM), _BF),
                jax.ShapeDtypeStruct((_NPAD, _DIM), _BF),
            ],
        )(h, ln1_g[l].reshape(1, _DIM), ln1_b[l].reshape(1, _DIM),
          wq[l].astype(_BF), bq[l].reshape(1, _DIM),
          wk[l].astype(_BF), bk[l].reshape(1, _DIM),
          wv[l].astype(_BF), bv[l].reshape(1, _DIM))

        drel = rel_bias[l].T

        h = pl.pallas_call(
            _layer_kernel,
            grid=(_NQT,),
            in_specs=[
                pl.BlockSpec((_TQ, _DIM), lambda i: (i, 0)),
                pl.BlockSpec((_NPAD, _DIM), lambda i: (0, 0)),
                pl.BlockSpec((_NPAD, _DIM), lambda i: (0, 0)),
                pl.BlockSpec((_TQ, _DIM), lambda i: (i, 0)),
                pl.BlockSpec((1, _TQ, 8), lambda i: (i, 0, 0)),
                pl.BlockSpec((8, _NPAD), lambda i: (0, 0)),
                pl.BlockSpec((_HEADS, _KB), lambda i: (0, 0)),
                pl.BlockSpec((_DIM, _DIM), lambda i: (0, 0)),
                pl.BlockSpec((1, _DIM), lambda i: (0, 0)),
                pl.BlockSpec((1, _DIM), lambda i: (0, 0)),
                pl.BlockSpec((1, _DIM), lambda i: (0, 0)),
                pl.BlockSpec((_DIM, _MLP), lambda i: (0, 0)),
                pl.BlockSpec((1, _MLP), lambda i: (0, 0)),
                pl.BlockSpec((_MLP, _DIM), lambda i: (0, 0)),
                pl.BlockSpec((1, _DIM), lambda i: (0, 0)),
            ],
            out_specs=pl.BlockSpec((_TQ, _DIM), lambda i: (i, 0)),
            out_shape=jax.ShapeDtypeStruct((_NPAD, _DIM), _F32),
        )(q, k, v, h, cq, ck, drel,
          wo[l].astype(_BF), bo[l].reshape(1, _DIM),
          ln2_g[l].reshape(1, _DIM), ln2_b[l].reshape(1, _DIM),
          w1[l].astype(_BF), b1[l].reshape(1, _MLP),
          w2[l].astype(_BF), b2[l].reshape(1, _DIM))

    hw = jnp.pad(head_w, ((0, 0), (0, 128 - _NC)))
    hb = jnp.pad(head_b, (0, 128 - _NC)).reshape(1, 128)
    out = pl.pallas_call(
        _final_kernel,
        grid=(1,),
        in_specs=[
            pl.BlockSpec((8, _DIM), lambda i: (0, 0)),
            pl.BlockSpec((1, _DIM), lambda i: (0, 0)),
            pl.BlockSpec((1, _DIM), lambda i: (0, 0)),
            pl.BlockSpec((_DIM, 128), lambda i: (0, 0)),
            pl.BlockSpec((1, 128), lambda i: (0, 0)),
        ],
        out_specs=pl.BlockSpec((1, 128), lambda i: (0, 0)),
        out_shape=jax.ShapeDtypeStruct((1, 128), _F32),
    )(h, lnf_g.reshape(1, _DIM), lnf_b.reshape(1, _DIM), hw, hb)

    return out[:, :_NC]


# speedup vs baseline: 98.9842x; 1.3088x over previous
import math

import jax
import jax.numpy as jnp
from jax.experimental import pallas as pl

_N = 2048
_IN_DIM = 768
_DIM = 512
_DEPTH = 2
_HEADS = 8
_KB = 10
_MLP = 2048
_NC = 2
_DH = _DIM // _HEADS

_NP1 = _N + 1
_TQ = 128
_NPAD = ((_NP1 + _TQ - 1) // _TQ) * _TQ
_NQT = _NPAD // _TQ
_NXT = _N // _TQ

_BF = jnp.bfloat16
_F32 = jnp.float32


def _lnorm(x, g, b, eps=1e-5):
    m = jnp.mean(x, axis=-1, keepdims=True)
    v = jnp.mean((x - m) ** 2, axis=-1, keepdims=True)
    return (x - m) / jnp.sqrt(v + eps) * g + b


def _embed_kernel(x_ref, w_ref, b_ref, o_ref):
    acc = jnp.dot(x_ref[...], w_ref[...], preferred_element_type=_F32)
    o_ref[...] = jax.nn.gelu(acc + b_ref[...])


def _qkv_kernel(h_ref, g_ref, b_ref, wq_ref, bq_ref, wk_ref, bk_ref,
                wv_ref, bv_ref, q_ref, k_ref, v_ref):
    scale = 1.0 / math.sqrt(_DH)
    xn = _lnorm(h_ref[...], g_ref[...], b_ref[...]).astype(_BF)
    q_ref[...] = ((jnp.dot(xn, wq_ref[...], preferred_element_type=_F32)
                   + bq_ref[...]) * scale).astype(_BF)
    k_ref[...] = (jnp.dot(xn, wk_ref[...], preferred_element_type=_F32)
                  + bk_ref[...]).astype(_BF)
    vf = (jnp.dot(xn, wv_ref[...], preferred_element_type=_F32)
          + bv_ref[...]).astype(_BF)
    ones = jnp.ones((vf.shape[0], 1), _BF)
    zeros = jnp.zeros((vf.shape[0], 128 - _DH - 1), _BF)
    slabs = []
    for hh in range(_HEADS):
        slabs.append(vf[:, hh * _DH:(hh + 1) * _DH])
        slabs.append(ones)
        slabs.append(zeros)
    v_ref[...] = jnp.concatenate(slabs, axis=1)


def _layer_kernel(q_ref, k_ref, v_ref, h_ref, cq_ref, ck_ref, drel_ref,
                  wo_ref, bo_ref, g2_ref, b2_ref, w1_ref, b1_ref,
                  w2_ref, b2b_ref, out_ref):
    qi = pl.program_id(0)

    qx = cq_ref[0, :, 0:1]
    qy = cq_ref[0, :, 1:2]
    kx = ck_ref[0:1, :]
    ky = ck_ref[1:2, :]
    dx = qx - kx
    dy = qy - ky
    d2 = (dx * dx + dy * dy).astype(_BF)
    masks = [d2 >= _BF((t / _KB) ** 2 - 1e-12) for t in range(1, _KB)]

    rows = qi * _TQ + jax.lax.broadcasted_iota(jnp.int32, (_TQ, 1), 0)
    cols = jax.lax.broadcasted_iota(jnp.int32, (1, _NPAD), 1)
    row0b = (rows == 0).astype(_BF)
    col0b = (cols == 0).astype(_BF)
    mask0 = (row0b + col0b) > _BF(0.5)
    kmask = jnp.where(cols > _N, -1e30, 0.0).astype(_BF)

    qf = q_ref[...]
    kf = k_ref[...]
    vf = v_ref[...]
    ohs = []
    for hh in range(_HEADS):
        sl = slice(hh * _DH, (hh + 1) * _DH)
        s = jax.lax.dot_general(qf[:, sl], kf[:, sl], (((1,), (1,)), ((), ())),
                                preferred_element_type=_F32).astype(_BF)
        r0 = drel_ref[hh:hh + 1, 0:1].astype(_BF)
        bias = jnp.broadcast_to(r0, s.shape)
        for t in range(1, _KB):
            rt = drel_ref[hh:hh + 1, t:t + 1].astype(_BF)
            bias = jnp.where(masks[t - 1], rt, bias)
        bias = jnp.where(mask0, _BF(0.0), bias)
        e = jnp.exp(s + bias + kmask)
        ohx = jnp.dot(e, vf[:, hh * 128:(hh + 1) * 128],
                      preferred_element_type=_F32)
        ohs.append(ohx[:, :_DH] * (1.0 / ohx[:, _DH:_DH + 1]))
    oc = jnp.concatenate(ohs, axis=1).astype(_BF)

    h2 = h_ref[...] + jnp.dot(oc, wo_ref[...],
                              preferred_element_type=_F32) + bo_ref[...]
    xn = _lnorm(h2, g2_ref[...], b2_ref[...]).astype(_BF)
    hh1 = jax.nn.gelu(jnp.dot(xn, w1_ref[...],
                              preferred_element_type=_F32) + b1_ref[...])
    ff = jnp.dot(hh1.astype(_BF), w2_ref[...],
                 preferred_element_type=_F32) + b2b_ref[...]
    out_ref[...] = h2 + ff


def _final_kernel(h_ref, g_ref, b_ref, w_ref, bh_ref, o_ref):
    x0 = h_ref[0:1, :]
    xn = _lnorm(x0, g_ref[...], b_ref[...])
    o_ref[...] = jnp.dot(xn, w_ref[...], preferred_element_type=_F32) + bh_ref[...]


def kernel(x, coord, lens, cls_token, fc_w, fc_b, ln1_g, ln1_b, wq, bq,
           wk, bk, wv, bv, wo, bo, rel_bias, ln2_g, ln2_b, w1, b1, w2, b2,
           lnf_g, lnf_b, head_w, head_b):
    x2 = x[0]

    emb = pl.pallas_call(
        _embed_kernel,
        grid=(_NXT,),
        in_specs=[
            pl.BlockSpec((_TQ, _IN_DIM), lambda i: (i, 0)),
            pl.BlockSpec((_IN_DIM, _DIM), lambda i: (0, 0)),
            pl.BlockSpec((1, _DIM), lambda i: (0, 0)),
        ],
        out_specs=pl.BlockSpec((_TQ, _DIM), lambda i: (i, 0)),
        out_shape=jax.ShapeDtypeStruct((_N, _DIM), _F32),
    )(x2, fc_w, fc_b.reshape(1, _DIM))

    h = jnp.concatenate([
        cls_token.reshape(1, _DIM),
        emb,
        jnp.zeros((_NPAD - _NP1, _DIM), _F32),
    ], axis=0)

    pc = jnp.concatenate([
        jnp.zeros((1, 2), _F32),
        coord[0],
        jnp.zeros((_NPAD - _NP1, 2), _F32),
    ], axis=0)
    cq = jnp.pad(pc, ((0, 0), (0, 6))).reshape(_NQT, _TQ, 8)
    ck = jnp.pad(pc.T, ((0, 6), (0, 0)))

    for l in range(_DEPTH):
        q, k, v = pl.pallas_call(
            _qkv_kernel,
            grid=(_NQT,),
            in_specs=[
                pl.BlockSpec((_TQ, _DIM), lambda i: (i, 0)),
                pl.BlockSpec((1, _DIM), lambda i: (0, 0)),
                pl.BlockSpec((1, _DIM), lambda i: (0, 0)),
                pl.BlockSpec((_DIM, _DIM), lambda i: (0, 0)),
                pl.BlockSpec((1, _DIM), lambda i: (0, 0)),
                pl.BlockSpec((_DIM, _DIM), lambda i: (0, 0)),
                pl.BlockSpec((1, _DIM), lambda i: (0, 0)),
                pl.BlockSpec((_DIM, _DIM), lambda i: (0, 0)),
                pl.BlockSpec((1, _DIM), lambda i: (0, 0)),
            ],
            out_specs=[
                pl.BlockSpec((_TQ, _DIM), lambda i: (i, 0)),
                pl.BlockSpec((_TQ, _DIM), lambda i: (i, 0)),
                pl.BlockSpec((_TQ, _HEADS * 128), lambda i: (i, 0)),
            ],
            out_shape=[
                jax.ShapeDtypeStruct((_NPAD, _DIM), _BF),
                jax.ShapeDtypeStruct((_NPAD, _DIM), _BF),
                jax.ShapeDtypeStruct((_NPAD, _HEADS * 128), _BF),
            ],
        )(h, ln1_g[l].reshape(1, _DIM), ln1_b[l].reshape(1, _DIM),
          wq[l].astype(_BF), bq[l].reshape(1, _DIM),
          wk[l].astype(_BF), bk[l].reshape(1, _DIM),
          wv[l].astype(_BF), bv[l].reshape(1, _DIM))

        drel = rel_bias[l].T

        h = pl.pallas_call(
            _layer_kernel,
            grid=(_NQT,),
            in_specs=[
                pl.BlockSpec((_TQ, _DIM), lambda i: (i, 0)),
                pl.BlockSpec((_NPAD, _DIM), lambda i: (0, 0)),
                pl.BlockSpec((_NPAD, _HEADS * 128), lambda i: (0, 0)),
                pl.BlockSpec((_TQ, _DIM), lambda i: (i, 0)),
                pl.BlockSpec((1, _TQ, 8), lambda i: (i, 0, 0)),
                pl.BlockSpec((8, _NPAD), lambda i: (0, 0)),
                pl.BlockSpec((_HEADS, _KB), lambda i: (0, 0)),
                pl.BlockSpec((_DIM, _DIM), lambda i: (0, 0)),
                pl.BlockSpec((1, _DIM), lambda i: (0, 0)),
                pl.BlockSpec((1, _DIM), lambda i: (0, 0)),
                pl.BlockSpec((1, _DIM), lambda i: (0, 0)),
                pl.BlockSpec((_DIM, _MLP), lambda i: (0, 0)),
                pl.BlockSpec((1, _MLP), lambda i: (0, 0)),
                pl.BlockSpec((_MLP, _DIM), lambda i: (0, 0)),
                pl.BlockSpec((1, _DIM), lambda i: (0, 0)),
            ],
            out_specs=pl.BlockSpec((_TQ, _DIM), lambda i: (i, 0)),
            out_shape=jax.ShapeDtypeStruct((_NPAD, _DIM), _F32),
        )(q, k, v, h, cq, ck, drel,
          wo[l].astype(_BF), bo[l].reshape(1, _DIM),
          ln2_g[l].reshape(1, _DIM), ln2_b[l].reshape(1, _DIM),
          w1[l].astype(_BF), b1[l].reshape(1, _MLP),
          w2[l].astype(_BF), b2[l].reshape(1, _DIM))

    hw = jnp.pad(head_w, ((0, 0), (0, 128 - _NC)))
    hb = jnp.pad(head_b, (0, 128 - _NC)).reshape(1, 128)
    out = pl.pallas_call(
        _final_kernel,
        grid=(1,),
        in_specs=[
            pl.BlockSpec((8, _DIM), lambda i: (0, 0)),
            pl.BlockSpec((1, _DIM), lambda i: (0, 0)),
            pl.BlockSpec((1, _DIM), lambda i: (0, 0)),
            pl.BlockSpec((_DIM, 128), lambda i: (0, 0)),
            pl.BlockSpec((1, 128), lambda i: (0, 0)),
        ],
        out_specs=pl.BlockSpec((1, 128), lambda i: (0, 0)),
        out_shape=jax.ShapeDtypeStruct((1, 128), _F32),
    )(h, lnf_g.reshape(1, _DIM), lnf_b.reshape(1, _DIM), hw, hb)

    return out[:, :_NC]
